# baseline (speedup 1.0000x reference)
import functools

import jax
import jax.numpy as jnp
from jax import lax
from jax.experimental import pallas as pl
from jax.experimental.pallas import tpu as pltpu
from jax.experimental.pallas import tpu_sc as plsc

_N, _D, _B, _DG = 100000, 128, 512, 64
_L = 16
_NS = 16
_NG = _N // _L
_GPT = _NG // _NS
_REM = _NG - _GPT * _NS
_NBG = 10
_BLK_ROWS = _NBG * _L
_BLK_ELEMS = _BLK_ROWS * _D
_RCH = 128
_NRCH = _B // _RCH


def _gather16(vec, idx):
    dnums = lax.GatherDimensionNumbers(
        offset_dims=(), collapsed_slice_dims=(0,), start_index_map=(0,))
    return lax.gather(vec, idx.reshape(_L, 1), dnums, (1,),
                      mode=lax.GatherScatterMode.PROMISE_IN_BOUNDS)


def _process(sid, f_hbm, ids_hbm, w_hbm, b_hbm, out_hbm,
             acc, inbufs, sems, idsbuf, wbuf, bbuf, idxv, shared):
    sg = _GPT * sid + jnp.minimum(sid, _REM)
    ng = jnp.where(sid < _REM, _GPT + 1, _GPT)
    nb = jnp.where(sid < _REM, _GPT // _NBG + 1, _GPT // _NBG)

    pltpu.sync_copy(ids_hbm.at[pl.ds(sg * _L, _GPT * _L)],
                    idsbuf.at[pl.ds(0, _GPT * _L)])

    @pl.when(sid < _REM)
    def _():
        pltpu.sync_copy(ids_hbm.at[pl.ds(sg * _L + _GPT * _L, _L * _NBG)],
                        idsbuf.at[pl.ds(_GPT * _L, _L * _NBG)])

    pltpu.sync_copy(w_hbm, wbuf)
    pltpu.sync_copy(b_hbm, bbuf)
    wc = [wbuf[pl.ds(_L * c, _L)] for c in range(_D // _L)]
    bv = bbuf[...]
    zero = jnp.zeros((_L,), jnp.float32)
    colv = [lax.iota(jnp.int32, _L) + _L * c for c in range(_D // _L)]
    bfly_idx = {k: lax.iota(jnp.int32, _L) ^ k for k in (1, 2, 4, 8)}

    @plsc.parallel_loop(0, _B, unroll=2)
    def _zrow(i):
        for c in range(_D // _L):
            acc[i, pl.ds(_L * c, _L)] = zero

    @pl.when(sid == 0)
    def _():
        pltpu.sync_copy(acc, shared)
    plsc.subcore_barrier()

    def _start(blk, buf, sem):
        row0 = (sg + blk * _NBG) * _L
        pltpu.async_copy(f_hbm.at[pl.ds(row0 * _D, _BLK_ELEMS)], buf, sem)

    def _proc(blk, buf):
        nrows = jnp.minimum(ng - blk * _NBG, _NBG) * _L

        @plsc.parallel_loop(0, nrows // 2, unroll=1)
        def _pair_rows(k):
            i = k * 2
            gidx = blk * _BLK_ROWS + i
            rid0 = plsc.load_gather(
                idsbuf, [jnp.full((_L,), gidx, jnp.int32)])
            rid1 = plsc.load_gather(
                idsbuf, [jnp.full((_L,), gidx + 1, jnp.int32)])
            base0 = i * _D
            base1 = base0 + _D

            def dot_partial(base):
                t = [buf[pl.ds(base + _L * c, _L)] * wc[c]
                     for c in range(_D // _L)]
                while len(t) > 1:
                    t = [t[j] + t[j + 1] for j in range(0, len(t), 2)]
                return t[0]

            p0 = dot_partial(base0)
            p1 = dot_partial(base1)
            m8 = (lax.iota(jnp.int32, _L) & 8) != 0
            su = p0 + _gather16(p0, bfly_idx[8])
            sv = p1 + _gather16(p1, bfly_idx[8])
            q = jnp.where(m8, sv, su)
            for bx in (bfly_idx[4], bfly_idx[2], bfly_idx[1]):
                q = q + _gather16(q, bx)
            w2 = 1.0 / (1.0 + jnp.exp(-(q + bv)))
            w0 = _gather16(w2, jnp.full((_L,), 0, jnp.int32))
            w1 = _gather16(w2, jnp.full((_L,), 8, jnp.int32))
            for c in range(_D // _L):
                v0 = buf[pl.ds(base0 + _L * c, _L)] * w0
                plsc.addupdate_scatter(acc, [rid0, colv[c]], v0)
            for c in range(_D // _L):
                v1 = buf[pl.ds(base1 + _L * c, _L)] * w1
                plsc.addupdate_scatter(acc, [rid1, colv[c]], v1)

    _start(0, inbufs[0], sems[0])

    @pl.when(nb > 1)
    def _():
        _start(1, inbufs[1], sems[1])

    def _pair(p, _):
        for b in range(2):
            blk = p * 2 + b

            @pl.when(blk < nb)
            def _():
                pltpu.make_async_copy(
                    f_hbm.at[pl.ds(0, _BLK_ELEMS)], inbufs[b],
                    sems[b]).wait()
                _proc(blk, inbufs[b])

                @pl.when(blk + 2 < nb)
                def _():
                    _start(blk + 2, inbufs[b], sems[b])
        return 0

    lax.fori_loop(0, (_GPT // _NBG + 2) // 2, _pair, 0)

    for j in range(_NRCH):
        pltpu.sync_copy(acc.at[pl.ds(j * _RCH, _RCH)],
                        shared.at[idxv[j]], add=True)
    plsc.subcore_barrier()

    rows = _B // _NS
    pltpu.sync_copy(shared.at[pl.ds(sid * rows, rows)],
                    out_hbm.at[pl.ds(sid * rows, rows)])


def _sc_body(fa, ia, fb, ib, wa, ba, wb, bb, idx_hbm, out_a, out_b,
             acc, inbuf0, inbuf1, sem0, sem1, idsbuf, wbuf, bbuf,
             idxv0, idxv1, idxv2, idxv3, shared):
    cid = lax.axis_index("c")
    sid = lax.axis_index("s")
    idxv = (idxv0, idxv1, idxv2, idxv3)
    for j in range(_NRCH):
        pltpu.sync_copy(idx_hbm.at[j], idxv[j])

    @pl.when(cid == 0)
    def _():
        _process(sid, fa, ia, wa, ba, out_a,
                 acc, (inbuf0, inbuf1), (sem0, sem1), idsbuf, wbuf, bbuf,
                 idxv, shared)

    @pl.when(cid == 1)
    def _():
        _process(sid, fb, ib, wb, bb, out_b,
                 acc, (inbuf0, inbuf1), (sem0, sem1), idsbuf, wbuf, bbuf,
                 idxv, shared)


def kernel(feats_atom, feats_bond, feats_global, batch_atom, batch_bond,
           W_atom, b_atom, W_bond, b_bond):
    fa = feats_atom.reshape(_N * _D)
    fb = feats_bond.reshape(_N * _D)
    ia = batch_atom.astype(jnp.int32)
    ib = batch_bond.astype(jnp.int32)
    wa = W_atom.reshape(_D)
    wb = W_bond.reshape(_D)
    ba = jnp.broadcast_to(b_atom.reshape(1), (_L,))
    bb = jnp.broadcast_to(b_bond.reshape(1), (_L,))
    idx = jnp.arange(_B, dtype=jnp.int32).reshape(_NRCH, _RCH)

    sc_fn = pl.kernel(
        _sc_body,
        out_type=(jax.ShapeDtypeStruct((_B, _D), jnp.float32),
                  jax.ShapeDtypeStruct((_B, _D), jnp.float32)),
        mesh=plsc.VectorSubcoreMesh(core_axis_name="c", subcore_axis_name="s"),
        compiler_params=pltpu.CompilerParams(needs_layout_passes=False),
        scratch_types=[
            pltpu.VMEM((_B, _D), jnp.float32),
            pltpu.VMEM((_BLK_ELEMS,), jnp.float32),
            pltpu.VMEM((_BLK_ELEMS,), jnp.float32),
            pltpu.SemaphoreType.DMA,
            pltpu.SemaphoreType.DMA,
            pltpu.VMEM(((_GPT + _NBG) * _L,), jnp.int32),
            pltpu.VMEM((_D,), jnp.float32),
            pltpu.VMEM((_L,), jnp.float32),
            pltpu.VMEM((_RCH,), jnp.int32),
            pltpu.VMEM((_RCH,), jnp.int32),
            pltpu.VMEM((_RCH,), jnp.int32),
            pltpu.VMEM((_RCH,), jnp.int32),
            pltpu.VMEM_SHARED((_B, _D), jnp.float32),
        ],
    )

    pa, pb = sc_fn(fa, ia, fb, ib, wa, ba, wb, bb, idx)
    return jnp.concatenate([pa, pb, feats_global], axis=-1)


# speedup vs baseline: 1.0499x; 1.0499x over previous
import functools

import jax
import jax.numpy as jnp
from jax import lax
from jax.experimental import pallas as pl
from jax.experimental.pallas import tpu as pltpu
from jax.experimental.pallas import tpu_sc as plsc

_N, _D, _B, _DG = 100000, 128, 512, 64
_L = 16
_NS = 16
_NG = _N // _L
_GPT = _NG // _NS
_REM = _NG - _GPT * _NS
_NBG = 10
_BLK_ROWS = _NBG * _L
_BLK_ELEMS = _BLK_ROWS * _D
_RCH = 128
_NRCH = _B // _RCH


def _gather16(vec, idx):
    dnums = lax.GatherDimensionNumbers(
        offset_dims=(), collapsed_slice_dims=(0,), start_index_map=(0,))
    return lax.gather(vec, idx.reshape(_L, 1), dnums, (1,),
                      mode=lax.GatherScatterMode.PROMISE_IN_BOUNDS)


def _process(sid, f_hbm, ids_hbm, w_hbm, b_hbm, out_hbm,
             acc, inbufs, sems, idsbuf, wbuf, bbuf, idxv, shared):
    sg = _GPT * sid + jnp.minimum(sid, _REM)
    ng = jnp.where(sid < _REM, _GPT + 1, _GPT)
    nb = jnp.where(sid < _REM, _GPT // _NBG + 1, _GPT // _NBG)

    pltpu.sync_copy(ids_hbm.at[pl.ds(sg * _L, _GPT * _L)],
                    idsbuf.at[pl.ds(0, _GPT * _L)])

    @pl.when(sid < _REM)
    def _():
        pltpu.sync_copy(ids_hbm.at[pl.ds(sg * _L + _GPT * _L, _L * _NBG)],
                        idsbuf.at[pl.ds(_GPT * _L, _L * _NBG)])

    pltpu.sync_copy(w_hbm, wbuf)
    pltpu.sync_copy(b_hbm, bbuf)
    wc = [wbuf[pl.ds(_L * c, _L)] for c in range(_D // _L)]
    bv = bbuf[...]
    zero = jnp.zeros((_L,), jnp.float32)
    colv = [lax.iota(jnp.int32, _L) + _L * c for c in range(_D // _L)]
    bfly_idx = {k: lax.iota(jnp.int32, _L) ^ k for k in (1, 2, 4, 8)}

    @plsc.parallel_loop(0, _B, unroll=2)
    def _zrow(i):
        for c in range(_D // _L):
            acc[i, pl.ds(_L * c, _L)] = zero

    @pl.when(sid == 0)
    def _():
        pltpu.sync_copy(acc, shared)
    plsc.subcore_barrier()

    def _start(blk, buf, sem):
        row0 = (sg + blk * _NBG) * _L
        pltpu.async_copy(f_hbm.at[pl.ds(row0 * _D, _BLK_ELEMS)], buf, sem)

    def _proc(blk, buf):
        nrows = jnp.minimum(ng - blk * _NBG, _NBG) * _L

        @plsc.parallel_loop(0, nrows, unroll=4)
        def _row(i):
            rid = plsc.load_gather(
                idsbuf, [jnp.full((_L,), blk * _BLK_ROWS + i, jnp.int32)])
            fbase = i * _D
            t = [buf[pl.ds(fbase + _L * c, _L)] * wc[c]
                 for c in range(_D // _L)]
            while len(t) > 1:
                t = [t[j] + t[j + 1] for j in range(0, len(t), 2)]
            p = t[0]
            for bx in bfly_idx.values():
                p = p + _gather16(p, bx)
            w = 1.0 / (1.0 + jnp.exp(-(p + bv)))
            for c in range(_D // _L):
                v = buf[pl.ds(fbase + _L * c, _L)] * w
                plsc.addupdate_scatter(acc, [rid, colv[c]], v)

    _start(0, inbufs[0], sems[0])

    @pl.when(nb > 1)
    def _():
        _start(1, inbufs[1], sems[1])

    def _pair(p, _):
        for b in range(2):
            blk = p * 2 + b

            @pl.when(blk < nb)
            def _():
                pltpu.make_async_copy(
                    f_hbm.at[pl.ds(0, _BLK_ELEMS)], inbufs[b],
                    sems[b]).wait()
                _proc(blk, inbufs[b])

                @pl.when(blk + 2 < nb)
                def _():
                    _start(blk + 2, inbufs[b], sems[b])
        return 0

    lax.fori_loop(0, (_GPT // _NBG + 2) // 2, _pair, 0)

    for j in range(_NRCH):
        pltpu.sync_copy(acc.at[pl.ds(j * _RCH, _RCH)],
                        shared.at[idxv[j]], add=True)
    plsc.subcore_barrier()

    rows = _B // _NS
    pltpu.sync_copy(shared.at[pl.ds(sid * rows, rows)],
                    out_hbm.at[pl.ds(sid * rows, rows)])


def _sc_body(fa, ia, fb, ib, wa, ba, wb, bb, idx_hbm, out_a, out_b,
             acc, inbuf0, inbuf1, sem0, sem1, idsbuf, wbuf, bbuf,
             idxv0, idxv1, idxv2, idxv3, shared):
    cid = lax.axis_index("c")
    sid = lax.axis_index("s")
    idxv = (idxv0, idxv1, idxv2, idxv3)
    for j in range(_NRCH):
        pltpu.sync_copy(idx_hbm.at[j], idxv[j])

    @pl.when(cid == 0)
    def _():
        _process(sid, fa, ia, wa, ba, out_a,
                 acc, (inbuf0, inbuf1), (sem0, sem1), idsbuf, wbuf, bbuf,
                 idxv, shared)

    @pl.when(cid == 1)
    def _():
        _process(sid, fb, ib, wb, bb, out_b,
                 acc, (inbuf0, inbuf1), (sem0, sem1), idsbuf, wbuf, bbuf,
                 idxv, shared)


def kernel(feats_atom, feats_bond, feats_global, batch_atom, batch_bond,
           W_atom, b_atom, W_bond, b_bond):
    fa = feats_atom.reshape(_N * _D)
    fb = feats_bond.reshape(_N * _D)
    ia = batch_atom.astype(jnp.int32)
    ib = batch_bond.astype(jnp.int32)
    wa = W_atom.reshape(_D)
    wb = W_bond.reshape(_D)
    ba = jnp.broadcast_to(b_atom.reshape(1), (_L,))
    bb = jnp.broadcast_to(b_bond.reshape(1), (_L,))
    idx = jnp.arange(_B, dtype=jnp.int32).reshape(_NRCH, _RCH)

    sc_fn = pl.kernel(
        _sc_body,
        out_type=(jax.ShapeDtypeStruct((_B, _D), jnp.float32),
                  jax.ShapeDtypeStruct((_B, _D), jnp.float32)),
        mesh=plsc.VectorSubcoreMesh(core_axis_name="c", subcore_axis_name="s"),
        compiler_params=pltpu.CompilerParams(needs_layout_passes=False),
        scratch_types=[
            pltpu.VMEM((_B, _D), jnp.float32),
            pltpu.VMEM((_BLK_ELEMS,), jnp.float32),
            pltpu.VMEM((_BLK_ELEMS,), jnp.float32),
            pltpu.SemaphoreType.DMA,
            pltpu.SemaphoreType.DMA,
            pltpu.VMEM(((_GPT + _NBG) * _L,), jnp.int32),
            pltpu.VMEM((_D,), jnp.float32),
            pltpu.VMEM((_L,), jnp.float32),
            pltpu.VMEM((_RCH,), jnp.int32),
            pltpu.VMEM((_RCH,), jnp.int32),
            pltpu.VMEM((_RCH,), jnp.int32),
            pltpu.VMEM((_RCH,), jnp.int32),
            pltpu.VMEM_SHARED((_B, _D), jnp.float32),
        ],
    )

    pa, pb = sc_fn(fa, ia, fb, ib, wa, ba, wb, bb, idx)
    return jnp.concatenate([pa, pb, feats_global], axis=-1)
